# baseline (speedup 1.0000x reference)
import functools

import jax
import jax.numpy as jnp
from jax import lax
from jax.experimental import pallas as pl
from jax.experimental.pallas import tpu as pltpu
from jax.experimental.pallas import tpu_sc as plsc

CH = 128
NBUF = 4


def _pq_body(x8_ref, wsb_ref, wrb_ref, p_ref, q_ref):
    x8 = x8_ref[...]
    p_ref[...] = jnp.dot(x8, wsb_ref[...], preferred_element_type=jnp.float32)
    q_ref[...] = jnp.dot(x8, wrb_ref[...], preferred_element_type=jnp.float32)


def _out_body(ea_ref, g_ref, wet_ref, bcol_ref, o_ref):
    acc = jnp.dot(wet_ref[...], ea_ref[...], preferred_element_type=jnp.float32)
    o_ref[...] = jnp.maximum(acc + g_ref[...] + bcol_ref[...], 0.0)


def _make_sc_gather_add(n_nodes, n_edges, d_out, n_workers):
    mesh = plsc.VectorSubcoreMesh(core_axis_name="c", subcore_axis_name="s")
    nchunks = n_edges // CH
    base_cnt = nchunks // n_workers
    rem = nchunks % n_workers
    tmax = -(-(base_cnt + (1 if rem else 0)) // NBUF) * NBUF
    sub_tiles = 16 // 8

    scratch_types = [
        pltpu.VMEM((NBUF, CH), jnp.int32),
        pltpu.VMEM((NBUF, CH), jnp.int32),
        pltpu.VMEM((NBUF, CH, d_out), jnp.float32),
        pltpu.VMEM((NBUF, CH, d_out), jnp.float32),
        pltpu.VMEM((NBUF, d_out, CH), jnp.float32),
    ] + [pltpu.SemaphoreType.DMA] * (3 * NBUF)

    @functools.partial(
        pl.kernel,
        out_type=jax.ShapeDtypeStruct((sub_tiles, nchunks, 8, 128), jnp.float32),
        mesh=mesh,
        scratch_types=scratch_types,
        compiler_params=pltpu.CompilerParams(
            use_tc_tiling_on_sc=False, needs_layout_passes=False),
    )
    def sc_kernel(p_hbm, q_hbm, ei3_hbm, gt_hbm,
                  sidx, didx, prow, qrow, obuf, *sems):
        semi = sems[:NBUF]
        semg = sems[NBUF:2 * NBUF]
        semo = sems[2 * NBUF:]
        wid = lax.axis_index("s") * 2 + lax.axis_index("c")
        cnt = base_cnt + (wid < rem).astype(jnp.int32)

        def chunk_id(t):
            return wid + n_workers * t

        def fire_idx(t, b):
            c = chunk_id(t)
            pltpu.make_async_copy(ei3_hbm.at[c, 0], sidx.at[b], semi[b]).start()
            pltpu.make_async_copy(ei3_hbm.at[c, 1], didx.at[b], semi[b]).start()

        def wait_idx(t, b):
            c = chunk_id(t)
            pltpu.make_async_copy(ei3_hbm.at[c, 0], sidx.at[b], semi[b]).wait()
            pltpu.make_async_copy(ei3_hbm.at[c, 1], didx.at[b], semi[b]).wait()

        def fire_gathers(b):
            pltpu.make_async_copy(
                p_hbm.at[sidx.at[b]], prow.at[b], semg[b]).start()
            pltpu.make_async_copy(
                q_hbm.at[didx.at[b]], qrow.at[b], semg[b]).start()

        def wait_gathers(b):
            pltpu.make_async_copy(
                p_hbm.at[sidx.at[b]], prow.at[b], semg[b]).wait()
            pltpu.make_async_copy(
                q_hbm.at[didx.at[b]], qrow.at[b], semg[b]).wait()

        def out_copies(t, b):
            c = chunk_id(t)
            return [
                pltpu.make_async_copy(
                    obuf.at[b, pl.ds(a * 8, 8), :],
                    gt_hbm.at[a, c],
                    semo[b],
                )
                for a in range(sub_tiles)
            ]

        for b in range(NBUF):
            fire_idx(b, b)
        for b in range(2):
            wait_idx(b, b)
            fire_gathers(b)

        ivec = lax.broadcasted_iota(jnp.int32, (16,), 0)
        zvec = jnp.zeros((16,), jnp.int32)

        def step(go, carry):
            for b in range(NBUF):
                t = go * NBUF + b

                @pl.when(t < cnt)
                def _():
                    wait_gathers(b)

                b2 = (b + 2) % NBUF

                @pl.when(t + 2 < cnt)
                def _():
                    wait_idx(t + 2, b2)
                    fire_gathers(b2)

                @pl.when(t + NBUF < cnt)
                def _():
                    fire_idx(t + NBUF, b)

                @pl.when(jnp.logical_and(t >= NBUF, t < cnt))
                def _():
                    for cp in out_copies(t - NBUF, b):
                        cp.wait()

                @pl.when(t < cnt)
                def _():
                    pb = prow.at[b]
                    qb = qrow.at[b]
                    ob = obuf.at[b]

                    @plsc.parallel_loop(0, CH, step=1, unroll=8)
                    def _(i):
                        row = pb[i, :] + qb[i, :]
                        plsc.store_scatter(ob, [ivec, zvec + i], row)

                    for cp in out_copies(t, b):
                        cp.start()
            return carry

        lax.fori_loop(0, tmax // NBUF, step, 0)
        for b in range(NBUF):
            t_last = cnt - NBUF + ((b - cnt) % NBUF)

            @pl.when(t_last >= 0)
            def _():
                for cp in out_copies(t_last, b):
                    cp.wait()

    return sc_kernel


def kernel(x, edge_index, edge_attr, W, b):
    n_nodes, d_in = x.shape
    n_edges, d_edge = edge_attr.shape
    d_out = W.shape[1]

    w_s = W[:d_in]
    w_r = W[d_in:2 * d_in]
    w_e = W[2 * d_in:]
    ei3 = edge_index.reshape(n_edges // CH, CH, 2).transpose(0, 2, 1)
    ea_t = edge_attr.T
    wet = w_e.T
    bcol = b.reshape(d_out, 1)

    pack = 128 // d_out
    x8 = x.reshape(n_nodes // pack, pack * d_in)
    eye = jnp.eye(pack, dtype=jnp.float32)
    wsb = jnp.kron(eye, w_s)
    wrb = jnp.kron(eye, w_r)
    p128, q128 = pl.pallas_call(
        _pq_body,
        out_shape=(
            jax.ShapeDtypeStruct((n_nodes // pack, 128), jnp.float32),
            jax.ShapeDtypeStruct((n_nodes // pack, 128), jnp.float32),
        ),
    )(x8, wsb, wrb)
    p = p128.reshape(n_nodes, d_out)
    q = q128.reshape(n_nodes, d_out)

    info = plsc.get_sparse_core_info()
    n_workers = info.num_cores * info.num_subcores
    assert n_edges % CH == 0
    g4 = _make_sc_gather_add(n_nodes, n_edges, d_out, n_workers)(p, q, ei3)
    gt = g4.transpose(0, 2, 1, 3).reshape(d_out, n_edges)

    blk = 32000
    grid = n_edges // blk
    out_t = pl.pallas_call(
        _out_body,
        grid=(grid,),
        in_specs=[
            pl.BlockSpec((d_edge, blk), lambda i: (0, i)),
            pl.BlockSpec((d_out, blk), lambda i: (0, i)),
            pl.BlockSpec((d_out, d_edge), lambda i: (0, 0)),
            pl.BlockSpec((d_out, 1), lambda i: (0, 0)),
        ],
        out_specs=pl.BlockSpec((d_out, blk), lambda i: (0, i)),
        out_shape=jax.ShapeDtypeStruct((d_out, n_edges), jnp.float32),
    )(ea_t, gt, wet, bcol)
    return out_t.T
